# Initial kernel scaffold; baseline (speedup 1.0000x reference)
import jax
import jax.numpy as jnp
from jax.experimental import pallas as pl


def kernel(nf, ef, gf, edge_index, batch, W_e, b_e, W_n, b_n, W_g, b_g):
    raise NotImplementedError("write your pallas kernel here")


# speedup vs baseline: 2.5550x; 2.5550x over previous
import functools

import jax
import jax.numpy as jnp
from jax import lax
from jax.experimental import pallas as pl
from jax.experimental.pallas import tpu as pltpu
from jax.experimental.pallas import tpu_sc as plsc

N = 10000
E = 320000
DF = 128
DE = 16
G = 32
DG = 32

NC = 2
NS = 16
NW = NC * NS
EPW = E // NW
CH = 80
NCHUNK = EPW // CH
RPT = 640
CW = 16
N8 = N // 8
N8P = 1256
FP32 = jnp.float32


def _tables_body(nf_ref, b2_ref, gf_ref, w1_ref, w2_ref, w4_ref, be_ref,
                 ps_ref, pd_ref):
    nf = nf_ref[...]
    oh = (b2_ref[...] == lax.broadcasted_iota(jnp.int32, (nf.shape[0], G), 1)
          ).astype(FP32)
    gfw = jnp.dot(gf_ref[...], w4_ref[...], preferred_element_type=FP32)
    ps_ref[...] = (jnp.dot(nf, w1_ref[...], preferred_element_type=FP32)
                   + jnp.dot(oh, gfw, preferred_element_type=FP32)
                   + be_ref[...])
    pd_ref[...] = jnp.dot(nf, w2_ref[...], preferred_element_type=FP32)


def _make_tables(nf, batch2, gf, w1, w2, w4, be2):
    bn = 1000
    grid = N // bn
    return pl.pallas_call(
        _tables_body,
        grid=(grid,),
        in_specs=[
            pl.BlockSpec((bn, DF), lambda i: (i, 0)),
            pl.BlockSpec((bn, 1), lambda i: (i, 0)),
            pl.BlockSpec((G, DG), lambda i: (0, 0)),
            pl.BlockSpec((DF, DF), lambda i: (0, 0)),
            pl.BlockSpec((DF, DF), lambda i: (0, 0)),
            pl.BlockSpec((DG, DF), lambda i: (0, 0)),
            pl.BlockSpec((1, DF), lambda i: (0, 0)),
        ],
        out_specs=[
            pl.BlockSpec((bn, DF), lambda i: (i, 0)),
            pl.BlockSpec((bn, DF), lambda i: (i, 0)),
        ],
        out_shape=[
            jax.ShapeDtypeStruct((N, DF), FP32),
            jax.ShapeDtypeStruct((N, DF), FP32),
        ],
    )(nf, batch2, gf, w1, w2, w4, be2)


def _pe_body(ef_ref, w3_ref, pe_ref):
    pe_ref[...] = jnp.dot(ef_ref[...], w3_ref[...], preferred_element_type=FP32)


def _make_pe(ef, w3):
    be = 4000
    grid = E // be
    return pl.pallas_call(
        _pe_body,
        grid=(grid,),
        in_specs=[
            pl.BlockSpec((be, DE), lambda i: (i, 0)),
            pl.BlockSpec((DE, DF), lambda i: (0, 0)),
        ],
        out_specs=pl.BlockSpec((be, DF), lambda i: (i, 0)),
        out_shape=jax.ShapeDtypeStruct((E, DF), FP32),
    )(ef, w3)


def _sc_edge_kernel(ps_hbm, pd_hbm, pe_hbm, src_hbm, dst_hbm,
                    mask8_hbm, z_hbm,
                    uef_hbm, accd_hbm, cnt8_hbm,
                    src_v, dst_v, dst8_v, dst7_v, a_v, b_v, c_v,
                    accd_s, cnt8_s,
                    sem0, sem1, sem2):
    c = lax.axis_index("c")
    s = lax.axis_index("s")
    wid = s * NC + c
    ebase = wid * EPW
    r0 = s * RPT

    @pl.when(s < NS - 1)
    def _():
        pltpu.sync_copy(z_hbm.at[pl.ds(r0, RPT)], accd_s.at[pl.ds(r0, RPT)])

    @pl.when(s == NS - 1)
    def _():
        t0 = (NS - 1) * RPT
        pltpu.sync_copy(z_hbm.at[pl.ds(t0, N - t0)],
                        accd_s.at[pl.ds(t0, N - t0)])

    @pl.when(s == 0)
    def _():
        pltpu.sync_copy(z_hbm.at[pl.ds(0, N8P)], cnt8_s)
    plsc.subcore_barrier()

    def chunk(i, carry):
        base = ebase + i * CH
        pltpu.sync_copy(src_hbm.at[pl.ds(base, CH)], src_v)
        pltpu.sync_copy(dst_hbm.at[pl.ds(base, CH)], dst_v)
        cps = pltpu.async_copy(ps_hbm.at[src_v], a_v, sem0)
        cpd = pltpu.async_copy(pd_hbm.at[dst_v], b_v, sem1)
        cpe = pltpu.async_copy(pe_hbm.at[pl.ds(base, CH)], c_v, sem2)

        for k in range(CH // 16):
            dk = pl.ds(k * 16, 16)
            d16 = dst_v[dk]
            dst8_v[dk] = jax.lax.shift_right_logical(d16, 3)
            dst7_v[dk] = jax.lax.bitwise_and(d16, 7)

        cps.wait()
        cpd.wait()
        cpe.wait()

        def vec(e, cy):
            for j in range(DF // 16):
                d = pl.ds(j * 16, 16)
                c_v[e, d] = jnp.maximum(a_v[e, d] + b_v[e, d] + c_v[e, d],
                                        0.0)
            return cy
        lax.fori_loop(0, CH, vec, 0)

        pltpu.sync_copy(c_v, uef_hbm.at[pl.ds(base, CH)])
        pltpu.async_copy(mask8_hbm.at[dst7_v], a_v, sem0).wait()
        pltpu.sync_copy(c_v, accd_s.at[dst_v], add=True)
        pltpu.sync_copy(a_v, cnt8_s.at[dst8_v], add=True)
        return carry
    lax.fori_loop(0, NCHUNK, chunk, 0)

    plsc.subcore_barrier()

    @pl.when(s < NS - 1)
    def _():
        pltpu.sync_copy(accd_s.at[pl.ds(r0, RPT)],
                        accd_hbm.at[pl.ds(c * N + r0, RPT)])
        pltpu.sync_copy(cnt8_s.at[pl.ds(s * 80, 80)],
                        cnt8_hbm.at[pl.ds(c * N8P + s * 80, 80)])

    @pl.when(s == NS - 1)
    def _():
        t0 = (NS - 1) * RPT
        pltpu.sync_copy(accd_s.at[pl.ds(t0, N - t0)],
                        accd_hbm.at[pl.ds(c * N + t0, N - t0)])
        pltpu.sync_copy(cnt8_s.at[pl.ds(1200, N8P - 1200)],
                        cnt8_hbm.at[pl.ds(c * N8P + 1200, N8P - 1200)])


def _sc_edge(ps, pd, pe, src, dst, mask8, z):
    mesh = plsc.VectorSubcoreMesh(core_axis_name="c", subcore_axis_name="s")
    f = pl.kernel(
        _sc_edge_kernel,
        out_type=[
            jax.ShapeDtypeStruct((E, DF), FP32),
            jax.ShapeDtypeStruct((NC * N, DF), FP32),
            jax.ShapeDtypeStruct((NC * N8P, DF), FP32),
        ],
        mesh=mesh,
        scratch_types=[
            pltpu.VMEM((CH,), jnp.int32),
            pltpu.VMEM((CH,), jnp.int32),
            pltpu.VMEM((CH,), jnp.int32),
            pltpu.VMEM((CH,), jnp.int32),
            pltpu.VMEM((CH, DF), FP32),
            pltpu.VMEM((CH, DF), FP32),
            pltpu.VMEM((CH, DF), FP32),
            pltpu.VMEM_SHARED((N, DF), FP32),
            pltpu.VMEM_SHARED((N8P, DF), FP32),
            pltpu.SemaphoreType.DMA,
            pltpu.SemaphoreType.DMA,
            pltpu.SemaphoreType.DMA,
        ],
    )
    return f(ps, pd, pe, src, dst, mask8, z)


def _edge_pool_body(src_ref, b2_ref, uef_ref, aggef_ref, cntg_ref,
                    agg_acc, cnt_acc, bndl_acc, bndu_acc):
    i = pl.program_id(0)
    ng = pl.num_programs(0)

    @pl.when(i == 0)
    def _():
        agg_acc[...] = jnp.zeros_like(agg_acc)
        cnt_acc[...] = jnp.zeros_like(cnt_acc)
        b = b2_ref[...]
        gi = lax.broadcasted_iota(jnp.int32, (b.shape[0], G), 1)
        bndl_acc[...] = jnp.sum((b < gi).astype(jnp.int32), axis=0,
                                keepdims=True)
        bndu_acc[...] = jnp.sum((b <= gi).astype(jnp.int32), axis=0,
                                keepdims=True)

    s2 = src_ref[...]
    oh = ((s2 >= bndl_acc[...]) & (s2 < bndu_acc[...])).astype(FP32)
    dn = (((0,), (0,)), ((), ()))
    agg_acc[...] += lax.dot_general(oh, uef_ref[...], dn,
                                    preferred_element_type=FP32)
    cnt_acc[...] += lax.dot_general(oh, jnp.ones((s2.shape[0], CW), FP32), dn,
                                    preferred_element_type=FP32)

    @pl.when(i == ng - 1)
    def _():
        aggef_ref[...] = agg_acc[...]
        cntg_ref[...] = cnt_acc[...]


def _edge_pool(src2, batch2, uef):
    be = 4000
    grid = E // be
    return pl.pallas_call(
        _edge_pool_body,
        grid=(grid,),
        in_specs=[
            pl.BlockSpec((be, 1), lambda i: (i, 0)),
            pl.BlockSpec((N, 1), lambda i: (0, 0)),
            pl.BlockSpec((be, DF), lambda i: (i, 0)),
        ],
        out_specs=[
            pl.BlockSpec((G, DF), lambda i: (0, 0)),
            pl.BlockSpec((G, CW), lambda i: (0, 0)),
        ],
        out_shape=[
            jax.ShapeDtypeStruct((G, DF), FP32),
            jax.ShapeDtypeStruct((G, CW), FP32),
        ],
        scratch_shapes=[
            pltpu.VMEM((G, DF), FP32),
            pltpu.VMEM((G, CW), FP32),
            pltpu.VMEM((1, G), jnp.int32),
            pltpu.VMEM((1, G), jnp.int32),
        ],
    )(src2, batch2, uef)


def _node_body(a0_ref, a1_ref, c0_ref, c1_ref, nf_ref, b2_ref, gf_ref,
               wn1_ref, wn2_ref, wn3_ref, bn_ref,
               agf_ref, cge_ref,
               wg1_ref, wg2_ref, wg3_ref, bg_ref,
               unf_ref, ugf_ref, aggn_acc, cntg_acc):
    i = pl.program_id(0)
    ng = pl.num_programs(0)

    @pl.when(i == 0)
    def _():
        aggn_acc[...] = jnp.zeros_like(aggn_acc)
        cntg_acc[...] = jnp.zeros_like(cntg_acc)

    nf = nf_ref[...]
    bn_rows = nf.shape[0]
    cnt = jnp.maximum(c0_ref[...] + c1_ref[...], 1.0)
    agg_e = (a0_ref[...] + a1_ref[...]) / cnt
    oh = (b2_ref[...] == lax.broadcasted_iota(jnp.int32, (bn_rows, G), 1)
          ).astype(FP32)
    gfw = jnp.dot(gf_ref[...], wn3_ref[...], preferred_element_type=FP32)
    unf = jax.nn.relu(
        jnp.dot(agg_e, wn1_ref[...], preferred_element_type=FP32)
        + jnp.dot(nf, wn2_ref[...], preferred_element_type=FP32)
        + jnp.dot(oh, gfw, preferred_element_type=FP32)
        + bn_ref[...]) + nf
    unf_ref[...] = unf

    dn = (((0,), (0,)), ((), ()))
    aggn_acc[...] += lax.dot_general(oh, unf, dn,
                                     preferred_element_type=FP32)
    cntg_acc[...] += lax.dot_general(oh, jnp.ones((bn_rows, DF), FP32), dn,
                                     preferred_element_type=FP32)

    @pl.when(i == ng - 1)
    def _():
        gf = gf_ref[...]
        agg_nf = aggn_acc[...] / jnp.maximum(cntg_acc[...], 1.0)
        cnt_eg = jnp.maximum(cge_ref[:, 0:1], 1.0)
        agg_ef = agf_ref[...] / cnt_eg
        ugf_ref[...] = jax.nn.relu(
            jnp.dot(agg_nf, wg1_ref[...], preferred_element_type=FP32)
            + jnp.dot(agg_ef, wg2_ref[...], preferred_element_type=FP32)
            + jnp.dot(gf, wg3_ref[...], preferred_element_type=FP32)
            + bg_ref[...]) + gf


def _node_global(accd, cntd, nf, batch2, gf, wn1, wn2, wn3, bn2,
                 aggef, cntg, wg1, wg2, wg3, bg2):
    bn = 1000
    grid = N // bn
    zero = lambda i: (0, 0)
    return pl.pallas_call(
        _node_body,
        grid=(grid,),
        in_specs=[
            pl.BlockSpec((bn, DF), lambda i: (i, 0)),
            pl.BlockSpec((bn, DF), lambda i: (i + grid, 0)),
            pl.BlockSpec((bn, 1), lambda i: (i, 0)),
            pl.BlockSpec((bn, 1), lambda i: (i + grid, 0)),
            pl.BlockSpec((bn, DF), lambda i: (i, 0)),
            pl.BlockSpec((bn, 1), lambda i: (i, 0)),
            pl.BlockSpec((G, DG), zero),
            pl.BlockSpec((DF, DF), zero),
            pl.BlockSpec((DF, DF), zero),
            pl.BlockSpec((DG, DF), zero),
            pl.BlockSpec((1, DF), zero),
            pl.BlockSpec((G, DF), zero),
            pl.BlockSpec((G, CW), zero),
            pl.BlockSpec((DF, DG), zero),
            pl.BlockSpec((DF, DG), zero),
            pl.BlockSpec((DG, DG), zero),
            pl.BlockSpec((1, DG), zero),
        ],
        out_specs=[
            pl.BlockSpec((bn, DF), lambda i: (i, 0)),
            pl.BlockSpec((G, DG), zero),
        ],
        out_shape=[
            jax.ShapeDtypeStruct((N, DF), FP32),
            jax.ShapeDtypeStruct((G, DG), FP32),
        ],
        scratch_shapes=[
            pltpu.VMEM((G, DF), FP32),
            pltpu.VMEM((G, DF), FP32),
        ],
    )(accd, accd, cntd, cntd, nf, batch2, gf, wn1, wn2, wn3, bn2,
      aggef, cntg, wg1, wg2, wg3, bg2)


def kernel(nf, ef, gf, edge_index, batch, W_e, b_e, W_n, b_n, W_g, b_g):
    src = edge_index[0].astype(jnp.int32)
    dst = edge_index[1].astype(jnp.int32)
    batch_i = batch.astype(jnp.int32)
    batch2 = batch_i[:, None]

    ps, pd = _make_tables(nf, batch2, gf,
                          W_e[:DF], W_e[DF:2 * DF], W_e[2 * DF + DE:],
                          b_e[None, :])
    pe = _make_pe(ef, W_e[2 * DF:2 * DF + DE])

    z = jnp.zeros((N, DF), FP32)
    mask8 = (lax.broadcasted_iota(jnp.int32, (8, DF), 1) // 16
             == lax.broadcasted_iota(jnp.int32, (8, DF), 0)).astype(FP32)
    uef, accd, cnt8 = _sc_edge(ps, pd, pe, src, dst, mask8, z)
    cntd = (cnt8.reshape(NC, N8P, 8, 16)[..., 0]
            .reshape(NC, N8P * 8)[:, :N].reshape(NC * N, 1))

    aggef, cntg = _edge_pool(src[:, None], batch2, uef)

    unf, ugf = _node_global(accd, cntd, nf, batch2, gf,
                            W_n[:DF], W_n[DF:2 * DF], W_n[2 * DF:],
                            b_n[None, :],
                            aggef, cntg,
                            W_g[:DF], W_g[DF:2 * DF], W_g[2 * DF:],
                            b_g[None, :])
    return unf, uef, ugf


# speedup vs baseline: 2.5619x; 1.0027x over previous
import functools

import jax
import jax.numpy as jnp
from jax import lax
from jax.experimental import pallas as pl
from jax.experimental.pallas import tpu as pltpu
from jax.experimental.pallas import tpu_sc as plsc

N = 10000
E = 320000
DF = 128
DE = 16
G = 32
DG = 32

NC = 2
NS = 16
NW = NC * NS
EPW = E // NW
CH = 80
NCHUNK = EPW // CH
RPT = 640
CW = 16
N8 = N // 8
N8P = 1256
FP32 = jnp.float32


def _tables_body(nf_ref, b2_ref, gf_ref, w1_ref, w2_ref, w4_ref, be_ref,
                 ps_ref, pd_ref):
    nf = nf_ref[...]
    oh = (b2_ref[...] == lax.broadcasted_iota(jnp.int32, (nf.shape[0], G), 1)
          ).astype(FP32)
    gfw = jnp.dot(gf_ref[...], w4_ref[...], preferred_element_type=FP32)
    ps_ref[...] = (jnp.dot(nf, w1_ref[...], preferred_element_type=FP32)
                   + jnp.dot(oh, gfw, preferred_element_type=FP32)
                   + be_ref[...])
    pd_ref[...] = jnp.dot(nf, w2_ref[...], preferred_element_type=FP32)


def _make_tables(nf, batch2, gf, w1, w2, w4, be2):
    bn = 1000
    grid = N // bn
    return pl.pallas_call(
        _tables_body,
        grid=(grid,),
        in_specs=[
            pl.BlockSpec((bn, DF), lambda i: (i, 0)),
            pl.BlockSpec((bn, 1), lambda i: (i, 0)),
            pl.BlockSpec((G, DG), lambda i: (0, 0)),
            pl.BlockSpec((DF, DF), lambda i: (0, 0)),
            pl.BlockSpec((DF, DF), lambda i: (0, 0)),
            pl.BlockSpec((DG, DF), lambda i: (0, 0)),
            pl.BlockSpec((1, DF), lambda i: (0, 0)),
        ],
        out_specs=[
            pl.BlockSpec((bn, DF), lambda i: (i, 0)),
            pl.BlockSpec((bn, DF), lambda i: (i, 0)),
        ],
        out_shape=[
            jax.ShapeDtypeStruct((N, DF), FP32),
            jax.ShapeDtypeStruct((N, DF), FP32),
        ],
    )(nf, batch2, gf, w1, w2, w4, be2)


def _pe_body(ef_ref, w3_ref, pe_ref):
    pe_ref[...] = jnp.dot(ef_ref[...], w3_ref[...], preferred_element_type=FP32)


def _make_pe(ef, w3):
    be = 4000
    grid = E // be
    return pl.pallas_call(
        _pe_body,
        grid=(grid,),
        in_specs=[
            pl.BlockSpec((be, DE), lambda i: (i, 0)),
            pl.BlockSpec((DE, DF), lambda i: (0, 0)),
        ],
        out_specs=pl.BlockSpec((be, DF), lambda i: (i, 0)),
        out_shape=jax.ShapeDtypeStruct((E, DF), FP32),
    )(ef, w3)


def _sc_edge_kernel(ps_hbm, pd_hbm, pe_hbm, src_hbm, dst_hbm,
                    mask8_hbm, z_hbm,
                    uef_hbm, accd_hbm, cnt8_hbm,
                    src_v, dst_v, dst8_v, dst7_v, a_v, b_v, c_v,
                    accd_s, cnt8_s,
                    sem0, sem1, sem2, sem3):
    c = lax.axis_index("c")
    s = lax.axis_index("s")
    wid = s * NC + c
    ebase = wid * EPW
    r0 = s * RPT

    @pl.when(s < NS - 1)
    def _():
        pltpu.sync_copy(z_hbm.at[pl.ds(r0, RPT)], accd_s.at[pl.ds(r0, RPT)])

    @pl.when(s == NS - 1)
    def _():
        t0 = (NS - 1) * RPT
        pltpu.sync_copy(z_hbm.at[pl.ds(t0, N - t0)],
                        accd_s.at[pl.ds(t0, N - t0)])

    @pl.when(s == 0)
    def _():
        pltpu.sync_copy(z_hbm.at[pl.ds(0, N8P)], cnt8_s)
    plsc.subcore_barrier()

    def chunk(i, carry):
        base = ebase + i * CH
        ci0 = pltpu.async_copy(src_hbm.at[pl.ds(base, CH)], src_v, sem0)
        ci1 = pltpu.async_copy(dst_hbm.at[pl.ds(base, CH)], dst_v, sem1)
        ci0.wait()
        ci1.wait()
        cps = pltpu.async_copy(ps_hbm.at[src_v], a_v, sem0)
        cpd = pltpu.async_copy(pd_hbm.at[dst_v], b_v, sem1)
        cpe = pltpu.async_copy(pe_hbm.at[pl.ds(base, CH)], c_v, sem2)

        for k in range(CH // 16):
            dk = pl.ds(k * 16, 16)
            d16 = dst_v[dk]
            dst8_v[dk] = jax.lax.shift_right_logical(d16, 3)
            dst7_v[dk] = jax.lax.bitwise_and(d16, 7)

        cps.wait()
        cpd.wait()
        cpe.wait()

        def vec(e, cy):
            for j in range(DF // 16):
                d = pl.ds(j * 16, 16)
                c_v[e, d] = jnp.maximum(a_v[e, d] + b_v[e, d] + c_v[e, d],
                                        0.0)
            return cy
        lax.fori_loop(0, CH, vec, 0)

        cw = pltpu.async_copy(c_v, uef_hbm.at[pl.ds(base, CH)], sem3)
        pltpu.async_copy(mask8_hbm.at[dst7_v], a_v, sem0).wait()
        pltpu.sync_copy(c_v, accd_s.at[dst_v], add=True)
        pltpu.sync_copy(a_v, cnt8_s.at[dst8_v], add=True)
        cw.wait()
        return carry
    lax.fori_loop(0, NCHUNK, chunk, 0)

    plsc.subcore_barrier()

    @pl.when(s < NS - 1)
    def _():
        pltpu.sync_copy(accd_s.at[pl.ds(r0, RPT)],
                        accd_hbm.at[pl.ds(c * N + r0, RPT)])
        pltpu.sync_copy(cnt8_s.at[pl.ds(s * 80, 80)],
                        cnt8_hbm.at[pl.ds(c * N8P + s * 80, 80)])

    @pl.when(s == NS - 1)
    def _():
        t0 = (NS - 1) * RPT
        pltpu.sync_copy(accd_s.at[pl.ds(t0, N - t0)],
                        accd_hbm.at[pl.ds(c * N + t0, N - t0)])
        pltpu.sync_copy(cnt8_s.at[pl.ds(1200, N8P - 1200)],
                        cnt8_hbm.at[pl.ds(c * N8P + 1200, N8P - 1200)])


def _sc_edge(ps, pd, pe, src, dst, mask8, z):
    mesh = plsc.VectorSubcoreMesh(core_axis_name="c", subcore_axis_name="s")
    f = pl.kernel(
        _sc_edge_kernel,
        out_type=[
            jax.ShapeDtypeStruct((E, DF), FP32),
            jax.ShapeDtypeStruct((NC * N, DF), FP32),
            jax.ShapeDtypeStruct((NC * N8P, DF), FP32),
        ],
        mesh=mesh,
        scratch_types=[
            pltpu.VMEM((CH,), jnp.int32),
            pltpu.VMEM((CH,), jnp.int32),
            pltpu.VMEM((CH,), jnp.int32),
            pltpu.VMEM((CH,), jnp.int32),
            pltpu.VMEM((CH, DF), FP32),
            pltpu.VMEM((CH, DF), FP32),
            pltpu.VMEM((CH, DF), FP32),
            pltpu.VMEM_SHARED((N, DF), FP32),
            pltpu.VMEM_SHARED((N8P, DF), FP32),
            pltpu.SemaphoreType.DMA,
            pltpu.SemaphoreType.DMA,
            pltpu.SemaphoreType.DMA,
            pltpu.SemaphoreType.DMA,
        ],
    )
    return f(ps, pd, pe, src, dst, mask8, z)


def _edge_pool_body(src_ref, b2_ref, uef_ref, aggef_ref, cntg_ref,
                    agg_acc, cnt_acc, bndl_acc, bndu_acc):
    i = pl.program_id(0)
    ng = pl.num_programs(0)

    @pl.when(i == 0)
    def _():
        agg_acc[...] = jnp.zeros_like(agg_acc)
        cnt_acc[...] = jnp.zeros_like(cnt_acc)
        b = b2_ref[...]
        gi = lax.broadcasted_iota(jnp.int32, (b.shape[0], G), 1)
        bndl_acc[...] = jnp.sum((b < gi).astype(jnp.int32), axis=0,
                                keepdims=True)
        bndu_acc[...] = jnp.sum((b <= gi).astype(jnp.int32), axis=0,
                                keepdims=True)

    s2 = src_ref[...]
    oh = ((s2 >= bndl_acc[...]) & (s2 < bndu_acc[...])).astype(FP32)
    dn = (((0,), (0,)), ((), ()))
    agg_acc[...] += lax.dot_general(oh, uef_ref[...], dn,
                                    preferred_element_type=FP32)
    cnt_acc[...] += lax.dot_general(oh, jnp.ones((s2.shape[0], CW), FP32), dn,
                                    preferred_element_type=FP32)

    @pl.when(i == ng - 1)
    def _():
        aggef_ref[...] = agg_acc[...]
        cntg_ref[...] = cnt_acc[...]


def _edge_pool(src2, batch2, uef):
    be = 4000
    grid = E // be
    return pl.pallas_call(
        _edge_pool_body,
        grid=(grid,),
        in_specs=[
            pl.BlockSpec((be, 1), lambda i: (i, 0)),
            pl.BlockSpec((N, 1), lambda i: (0, 0)),
            pl.BlockSpec((be, DF), lambda i: (i, 0)),
        ],
        out_specs=[
            pl.BlockSpec((G, DF), lambda i: (0, 0)),
            pl.BlockSpec((G, CW), lambda i: (0, 0)),
        ],
        out_shape=[
            jax.ShapeDtypeStruct((G, DF), FP32),
            jax.ShapeDtypeStruct((G, CW), FP32),
        ],
        scratch_shapes=[
            pltpu.VMEM((G, DF), FP32),
            pltpu.VMEM((G, CW), FP32),
            pltpu.VMEM((1, G), jnp.int32),
            pltpu.VMEM((1, G), jnp.int32),
        ],
    )(src2, batch2, uef)


def _node_body(a0_ref, a1_ref, c0_ref, c1_ref, nf_ref, b2_ref, gf_ref,
               wn1_ref, wn2_ref, wn3_ref, bn_ref,
               agf_ref, cge_ref,
               wg1_ref, wg2_ref, wg3_ref, bg_ref,
               unf_ref, ugf_ref, aggn_acc, cntg_acc):
    i = pl.program_id(0)
    ng = pl.num_programs(0)

    @pl.when(i == 0)
    def _():
        aggn_acc[...] = jnp.zeros_like(aggn_acc)
        cntg_acc[...] = jnp.zeros_like(cntg_acc)

    nf = nf_ref[...]
    bn_rows = nf.shape[0]
    cnt = jnp.maximum(c0_ref[...] + c1_ref[...], 1.0)
    agg_e = (a0_ref[...] + a1_ref[...]) / cnt
    oh = (b2_ref[...] == lax.broadcasted_iota(jnp.int32, (bn_rows, G), 1)
          ).astype(FP32)
    gfw = jnp.dot(gf_ref[...], wn3_ref[...], preferred_element_type=FP32)
    unf = jax.nn.relu(
        jnp.dot(agg_e, wn1_ref[...], preferred_element_type=FP32)
        + jnp.dot(nf, wn2_ref[...], preferred_element_type=FP32)
        + jnp.dot(oh, gfw, preferred_element_type=FP32)
        + bn_ref[...]) + nf
    unf_ref[...] = unf

    dn = (((0,), (0,)), ((), ()))
    aggn_acc[...] += lax.dot_general(oh, unf, dn,
                                     preferred_element_type=FP32)
    cntg_acc[...] += lax.dot_general(oh, jnp.ones((bn_rows, DF), FP32), dn,
                                     preferred_element_type=FP32)

    @pl.when(i == ng - 1)
    def _():
        gf = gf_ref[...]
        agg_nf = aggn_acc[...] / jnp.maximum(cntg_acc[...], 1.0)
        cnt_eg = jnp.maximum(cge_ref[:, 0:1], 1.0)
        agg_ef = agf_ref[...] / cnt_eg
        ugf_ref[...] = jax.nn.relu(
            jnp.dot(agg_nf, wg1_ref[...], preferred_element_type=FP32)
            + jnp.dot(agg_ef, wg2_ref[...], preferred_element_type=FP32)
            + jnp.dot(gf, wg3_ref[...], preferred_element_type=FP32)
            + bg_ref[...]) + gf


def _node_global(accd, cntd, nf, batch2, gf, wn1, wn2, wn3, bn2,
                 aggef, cntg, wg1, wg2, wg3, bg2):
    bn = 1000
    grid = N // bn
    zero = lambda i: (0, 0)
    return pl.pallas_call(
        _node_body,
        grid=(grid,),
        in_specs=[
            pl.BlockSpec((bn, DF), lambda i: (i, 0)),
            pl.BlockSpec((bn, DF), lambda i: (i + grid, 0)),
            pl.BlockSpec((bn, 1), lambda i: (i, 0)),
            pl.BlockSpec((bn, 1), lambda i: (i + grid, 0)),
            pl.BlockSpec((bn, DF), lambda i: (i, 0)),
            pl.BlockSpec((bn, 1), lambda i: (i, 0)),
            pl.BlockSpec((G, DG), zero),
            pl.BlockSpec((DF, DF), zero),
            pl.BlockSpec((DF, DF), zero),
            pl.BlockSpec((DG, DF), zero),
            pl.BlockSpec((1, DF), zero),
            pl.BlockSpec((G, DF), zero),
            pl.BlockSpec((G, CW), zero),
            pl.BlockSpec((DF, DG), zero),
            pl.BlockSpec((DF, DG), zero),
            pl.BlockSpec((DG, DG), zero),
            pl.BlockSpec((1, DG), zero),
        ],
        out_specs=[
            pl.BlockSpec((bn, DF), lambda i: (i, 0)),
            pl.BlockSpec((G, DG), zero),
        ],
        out_shape=[
            jax.ShapeDtypeStruct((N, DF), FP32),
            jax.ShapeDtypeStruct((G, DG), FP32),
        ],
        scratch_shapes=[
            pltpu.VMEM((G, DF), FP32),
            pltpu.VMEM((G, DF), FP32),
        ],
    )(accd, accd, cntd, cntd, nf, batch2, gf, wn1, wn2, wn3, bn2,
      aggef, cntg, wg1, wg2, wg3, bg2)


def kernel(nf, ef, gf, edge_index, batch, W_e, b_e, W_n, b_n, W_g, b_g):
    src = edge_index[0].astype(jnp.int32)
    dst = edge_index[1].astype(jnp.int32)
    batch_i = batch.astype(jnp.int32)
    batch2 = batch_i[:, None]

    ps, pd = _make_tables(nf, batch2, gf,
                          W_e[:DF], W_e[DF:2 * DF], W_e[2 * DF + DE:],
                          b_e[None, :])
    pe = _make_pe(ef, W_e[2 * DF:2 * DF + DE])

    z = jnp.zeros((N, DF), FP32)
    mask8 = (lax.broadcasted_iota(jnp.int32, (8, DF), 1) // 16
             == lax.broadcasted_iota(jnp.int32, (8, DF), 0)).astype(FP32)
    uef, accd, cnt8 = _sc_edge(ps, pd, pe, src, dst, mask8, z)
    cntd = (cnt8.reshape(NC, N8P, 8, 16)[..., 0]
            .reshape(NC, N8P * 8)[:, :N].reshape(NC * N, 1))

    aggef, cntg = _edge_pool(src[:, None], batch2, uef)

    unf, ugf = _node_global(accd, cntd, nf, batch2, gf,
                            W_n[:DF], W_n[DF:2 * DF], W_n[2 * DF:],
                            b_n[None, :],
                            aggef, cntg,
                            W_g[:DF], W_g[DF:2 * DF], W_g[2 * DF:],
                            b_g[None, :])
    return unf, uef, ugf


# speedup vs baseline: 2.5629x; 1.0004x over previous
import functools

import jax
import jax.numpy as jnp
from jax import lax
from jax.experimental import pallas as pl
from jax.experimental.pallas import tpu as pltpu
from jax.experimental.pallas import tpu_sc as plsc

N = 10000
E = 320000
DF = 128
DE = 16
G = 32
DG = 32

NC = 2
NS = 16
NW = NC * NS
EPW = E // NW
CH = 80
NCHUNK = EPW // CH
RPT = 640
CW = 16
N8 = N // 8
N8P = 1256
FP32 = jnp.float32


def _tables_body(nf_ref, b2_ref, gf_ref, w1_ref, w2_ref, w4_ref, be_ref,
                 ps_ref, pd_ref):
    nf = nf_ref[...]
    oh = (b2_ref[...] == lax.broadcasted_iota(jnp.int32, (nf.shape[0], G), 1)
          ).astype(FP32)
    gfw = jnp.dot(gf_ref[...], w4_ref[...], preferred_element_type=FP32)
    ps_ref[...] = (jnp.dot(nf, w1_ref[...], preferred_element_type=FP32)
                   + jnp.dot(oh, gfw, preferred_element_type=FP32)
                   + be_ref[...])
    pd_ref[...] = jnp.dot(nf, w2_ref[...], preferred_element_type=FP32)


def _make_tables(nf, batch2, gf, w1, w2, w4, be2):
    bn = 1000
    grid = N // bn
    return pl.pallas_call(
        _tables_body,
        grid=(grid,),
        in_specs=[
            pl.BlockSpec((bn, DF), lambda i: (i, 0)),
            pl.BlockSpec((bn, 1), lambda i: (i, 0)),
            pl.BlockSpec((G, DG), lambda i: (0, 0)),
            pl.BlockSpec((DF, DF), lambda i: (0, 0)),
            pl.BlockSpec((DF, DF), lambda i: (0, 0)),
            pl.BlockSpec((DG, DF), lambda i: (0, 0)),
            pl.BlockSpec((1, DF), lambda i: (0, 0)),
        ],
        out_specs=[
            pl.BlockSpec((bn, DF), lambda i: (i, 0)),
            pl.BlockSpec((bn, DF), lambda i: (i, 0)),
        ],
        out_shape=[
            jax.ShapeDtypeStruct((N, DF), FP32),
            jax.ShapeDtypeStruct((N, DF), FP32),
        ],
    )(nf, batch2, gf, w1, w2, w4, be2)


def _pe_body(ef_ref, w3_ref, pe_ref):
    pe_ref[...] = jnp.dot(ef_ref[...], w3_ref[...], preferred_element_type=FP32)


def _make_pe(ef, w3):
    be = 4000
    grid = E // be
    return pl.pallas_call(
        _pe_body,
        grid=(grid,),
        in_specs=[
            pl.BlockSpec((be, DE), lambda i: (i, 0)),
            pl.BlockSpec((DE, DF), lambda i: (0, 0)),
        ],
        out_specs=pl.BlockSpec((be, DF), lambda i: (i, 0)),
        out_shape=jax.ShapeDtypeStruct((E, DF), FP32),
    )(ef, w3)


def _sc_edge_kernel(ps_hbm, pd_hbm, pe_hbm, src_hbm, dst_hbm,
                    mask8_hbm, z_hbm,
                    uef_hbm, accd_hbm, cnt8_hbm,
                    src_v, dst_v, dst8_v, dst7_v, a_v, b_v, c_v,
                    accd_s, cnt8_s,
                    sem0, sem1, sem2, sem3):
    c = lax.axis_index("c")
    s = lax.axis_index("s")
    wid = s * NC + c
    ebase = wid * EPW
    r0 = s * RPT

    @pl.when(s < NS - 1)
    def _():
        pltpu.sync_copy(z_hbm.at[pl.ds(r0, RPT)], accd_s.at[pl.ds(r0, RPT)])

    @pl.when(s == NS - 1)
    def _():
        t0 = (NS - 1) * RPT
        pltpu.sync_copy(z_hbm.at[pl.ds(t0, N - t0)],
                        accd_s.at[pl.ds(t0, N - t0)])

    @pl.when(s == 0)
    def _():
        pltpu.sync_copy(z_hbm.at[pl.ds(0, N8P)], cnt8_s)
    plsc.subcore_barrier()

    def chunk(i, carry):
        base = ebase + i * CH
        ci0 = pltpu.async_copy(src_hbm.at[pl.ds(base, CH)], src_v, sem0)
        ci1 = pltpu.async_copy(dst_hbm.at[pl.ds(base, CH)], dst_v, sem1)
        ci0.wait()
        ci1.wait()
        cps = pltpu.async_copy(ps_hbm.at[src_v], a_v, sem0)
        cpd = pltpu.async_copy(pd_hbm.at[dst_v], b_v, sem1)
        cpe = pltpu.async_copy(pe_hbm.at[pl.ds(base, CH)], c_v, sem2)

        for k in range(CH // 16):
            dk = pl.ds(k * 16, 16)
            d16 = dst_v[dk]
            dst8_v[dk] = jax.lax.shift_right_logical(d16, 3)
            dst7_v[dk] = jax.lax.bitwise_and(d16, 7)

        cps.wait()
        cpd.wait()
        cpe.wait()

        def vec(e, cy):
            for j in range(DF // 16):
                d = pl.ds(j * 16, 16)
                c_v[e, d] = jnp.maximum(a_v[e, d] + b_v[e, d] + c_v[e, d],
                                        0.0)
            return cy
        lax.fori_loop(0, CH, vec, 0)

        cw = pltpu.async_copy(c_v, uef_hbm.at[pl.ds(base, CH)], sem3)
        cm = pltpu.async_copy(mask8_hbm.at[dst7_v], a_v, sem0)
        ca = pltpu.async_copy(c_v, accd_s.at[dst_v], sem2, add=True)
        cm.wait()
        cb = pltpu.async_copy(a_v, cnt8_s.at[dst8_v], sem1, add=True)
        ca.wait()
        cb.wait()
        cw.wait()
        return carry
    lax.fori_loop(0, NCHUNK, chunk, 0)

    plsc.subcore_barrier()

    @pl.when(s < NS - 1)
    def _():
        pltpu.sync_copy(accd_s.at[pl.ds(r0, RPT)],
                        accd_hbm.at[pl.ds(c * N + r0, RPT)])
        pltpu.sync_copy(cnt8_s.at[pl.ds(s * 80, 80)],
                        cnt8_hbm.at[pl.ds(c * N8P + s * 80, 80)])

    @pl.when(s == NS - 1)
    def _():
        t0 = (NS - 1) * RPT
        pltpu.sync_copy(accd_s.at[pl.ds(t0, N - t0)],
                        accd_hbm.at[pl.ds(c * N + t0, N - t0)])
        pltpu.sync_copy(cnt8_s.at[pl.ds(1200, N8P - 1200)],
                        cnt8_hbm.at[pl.ds(c * N8P + 1200, N8P - 1200)])


def _sc_edge(ps, pd, pe, src, dst, mask8, z):
    mesh = plsc.VectorSubcoreMesh(core_axis_name="c", subcore_axis_name="s")
    f = pl.kernel(
        _sc_edge_kernel,
        out_type=[
            jax.ShapeDtypeStruct((E, DF), FP32),
            jax.ShapeDtypeStruct((NC * N, DF), FP32),
            jax.ShapeDtypeStruct((NC * N8P, DF), FP32),
        ],
        mesh=mesh,
        scratch_types=[
            pltpu.VMEM((CH,), jnp.int32),
            pltpu.VMEM((CH,), jnp.int32),
            pltpu.VMEM((CH,), jnp.int32),
            pltpu.VMEM((CH,), jnp.int32),
            pltpu.VMEM((CH, DF), FP32),
            pltpu.VMEM((CH, DF), FP32),
            pltpu.VMEM((CH, DF), FP32),
            pltpu.VMEM_SHARED((N, DF), FP32),
            pltpu.VMEM_SHARED((N8P, DF), FP32),
            pltpu.SemaphoreType.DMA,
            pltpu.SemaphoreType.DMA,
            pltpu.SemaphoreType.DMA,
            pltpu.SemaphoreType.DMA,
        ],
    )
    return f(ps, pd, pe, src, dst, mask8, z)


def _edge_pool_body(src_ref, b2_ref, uef_ref, aggef_ref, cntg_ref,
                    agg_acc, cnt_acc, bndl_acc, bndu_acc):
    i = pl.program_id(0)
    ng = pl.num_programs(0)

    @pl.when(i == 0)
    def _():
        agg_acc[...] = jnp.zeros_like(agg_acc)
        cnt_acc[...] = jnp.zeros_like(cnt_acc)
        b = b2_ref[...]
        gi = lax.broadcasted_iota(jnp.int32, (b.shape[0], G), 1)
        bndl_acc[...] = jnp.sum((b < gi).astype(jnp.int32), axis=0,
                                keepdims=True)
        bndu_acc[...] = jnp.sum((b <= gi).astype(jnp.int32), axis=0,
                                keepdims=True)

    s2 = src_ref[...]
    oh = ((s2 >= bndl_acc[...]) & (s2 < bndu_acc[...])).astype(FP32)
    dn = (((0,), (0,)), ((), ()))
    agg_acc[...] += lax.dot_general(oh, uef_ref[...], dn,
                                    preferred_element_type=FP32)
    cnt_acc[...] += lax.dot_general(oh, jnp.ones((s2.shape[0], CW), FP32), dn,
                                    preferred_element_type=FP32)

    @pl.when(i == ng - 1)
    def _():
        aggef_ref[...] = agg_acc[...]
        cntg_ref[...] = cnt_acc[...]


def _edge_pool(src2, batch2, uef):
    be = 4000
    grid = E // be
    return pl.pallas_call(
        _edge_pool_body,
        grid=(grid,),
        in_specs=[
            pl.BlockSpec((be, 1), lambda i: (i, 0)),
            pl.BlockSpec((N, 1), lambda i: (0, 0)),
            pl.BlockSpec((be, DF), lambda i: (i, 0)),
        ],
        out_specs=[
            pl.BlockSpec((G, DF), lambda i: (0, 0)),
            pl.BlockSpec((G, CW), lambda i: (0, 0)),
        ],
        out_shape=[
            jax.ShapeDtypeStruct((G, DF), FP32),
            jax.ShapeDtypeStruct((G, CW), FP32),
        ],
        scratch_shapes=[
            pltpu.VMEM((G, DF), FP32),
            pltpu.VMEM((G, CW), FP32),
            pltpu.VMEM((1, G), jnp.int32),
            pltpu.VMEM((1, G), jnp.int32),
        ],
    )(src2, batch2, uef)


def _node_body(a0_ref, a1_ref, c0_ref, c1_ref, nf_ref, b2_ref, gf_ref,
               wn1_ref, wn2_ref, wn3_ref, bn_ref,
               agf_ref, cge_ref,
               wg1_ref, wg2_ref, wg3_ref, bg_ref,
               unf_ref, ugf_ref, aggn_acc, cntg_acc):
    i = pl.program_id(0)
    ng = pl.num_programs(0)

    @pl.when(i == 0)
    def _():
        aggn_acc[...] = jnp.zeros_like(aggn_acc)
        cntg_acc[...] = jnp.zeros_like(cntg_acc)

    nf = nf_ref[...]
    bn_rows = nf.shape[0]
    cnt = jnp.maximum(c0_ref[...] + c1_ref[...], 1.0)
    agg_e = (a0_ref[...] + a1_ref[...]) / cnt
    oh = (b2_ref[...] == lax.broadcasted_iota(jnp.int32, (bn_rows, G), 1)
          ).astype(FP32)
    gfw = jnp.dot(gf_ref[...], wn3_ref[...], preferred_element_type=FP32)
    unf = jax.nn.relu(
        jnp.dot(agg_e, wn1_ref[...], preferred_element_type=FP32)
        + jnp.dot(nf, wn2_ref[...], preferred_element_type=FP32)
        + jnp.dot(oh, gfw, preferred_element_type=FP32)
        + bn_ref[...]) + nf
    unf_ref[...] = unf

    dn = (((0,), (0,)), ((), ()))
    aggn_acc[...] += lax.dot_general(oh, unf, dn,
                                     preferred_element_type=FP32)
    cntg_acc[...] += lax.dot_general(oh, jnp.ones((bn_rows, DF), FP32), dn,
                                     preferred_element_type=FP32)

    @pl.when(i == ng - 1)
    def _():
        gf = gf_ref[...]
        agg_nf = aggn_acc[...] / jnp.maximum(cntg_acc[...], 1.0)
        cnt_eg = jnp.maximum(cge_ref[:, 0:1], 1.0)
        agg_ef = agf_ref[...] / cnt_eg
        ugf_ref[...] = jax.nn.relu(
            jnp.dot(agg_nf, wg1_ref[...], preferred_element_type=FP32)
            + jnp.dot(agg_ef, wg2_ref[...], preferred_element_type=FP32)
            + jnp.dot(gf, wg3_ref[...], preferred_element_type=FP32)
            + bg_ref[...]) + gf


def _node_global(accd, cntd, nf, batch2, gf, wn1, wn2, wn3, bn2,
                 aggef, cntg, wg1, wg2, wg3, bg2):
    bn = 1000
    grid = N // bn
    zero = lambda i: (0, 0)
    return pl.pallas_call(
        _node_body,
        grid=(grid,),
        in_specs=[
            pl.BlockSpec((bn, DF), lambda i: (i, 0)),
            pl.BlockSpec((bn, DF), lambda i: (i + grid, 0)),
            pl.BlockSpec((bn, 1), lambda i: (i, 0)),
            pl.BlockSpec((bn, 1), lambda i: (i + grid, 0)),
            pl.BlockSpec((bn, DF), lambda i: (i, 0)),
            pl.BlockSpec((bn, 1), lambda i: (i, 0)),
            pl.BlockSpec((G, DG), zero),
            pl.BlockSpec((DF, DF), zero),
            pl.BlockSpec((DF, DF), zero),
            pl.BlockSpec((DG, DF), zero),
            pl.BlockSpec((1, DF), zero),
            pl.BlockSpec((G, DF), zero),
            pl.BlockSpec((G, CW), zero),
            pl.BlockSpec((DF, DG), zero),
            pl.BlockSpec((DF, DG), zero),
            pl.BlockSpec((DG, DG), zero),
            pl.BlockSpec((1, DG), zero),
        ],
        out_specs=[
            pl.BlockSpec((bn, DF), lambda i: (i, 0)),
            pl.BlockSpec((G, DG), zero),
        ],
        out_shape=[
            jax.ShapeDtypeStruct((N, DF), FP32),
            jax.ShapeDtypeStruct((G, DG), FP32),
        ],
        scratch_shapes=[
            pltpu.VMEM((G, DF), FP32),
            pltpu.VMEM((G, DF), FP32),
        ],
    )(accd, accd, cntd, cntd, nf, batch2, gf, wn1, wn2, wn3, bn2,
      aggef, cntg, wg1, wg2, wg3, bg2)


def kernel(nf, ef, gf, edge_index, batch, W_e, b_e, W_n, b_n, W_g, b_g):
    src = edge_index[0].astype(jnp.int32)
    dst = edge_index[1].astype(jnp.int32)
    batch_i = batch.astype(jnp.int32)
    batch2 = batch_i[:, None]

    ps, pd = _make_tables(nf, batch2, gf,
                          W_e[:DF], W_e[DF:2 * DF], W_e[2 * DF + DE:],
                          b_e[None, :])
    pe = _make_pe(ef, W_e[2 * DF:2 * DF + DE])

    z = jnp.zeros((N, DF), FP32)
    mask8 = (lax.broadcasted_iota(jnp.int32, (8, DF), 1) // 16
             == lax.broadcasted_iota(jnp.int32, (8, DF), 0)).astype(FP32)
    uef, accd, cnt8 = _sc_edge(ps, pd, pe, src, dst, mask8, z)
    cntd = (cnt8.reshape(NC, N8P, 8, 16)[..., 0]
            .reshape(NC, N8P * 8)[:, :N].reshape(NC * N, 1))

    aggef, cntg = _edge_pool(src[:, None], batch2, uef)

    unf, ugf = _node_global(accd, cntd, nf, batch2, gf,
                            W_n[:DF], W_n[DF:2 * DF], W_n[2 * DF:],
                            b_n[None, :],
                            aggef, cntg,
                            W_g[:DF], W_g[DF:2 * DF], W_g[2 * DF:],
                            b_g[None, :])
    return unf, uef, ugf
